# baseline (speedup 1.0000x reference)
import functools

import jax
import jax.numpy as jnp
from jax import lax
from jax.experimental import pallas as pl
from jax.experimental.pallas import tpu as pltpu
from jax.experimental.pallas import tpu_sc as plsc

NN = 10000
EE = 320000
DD = 128
EPS = 1e-8
NP = 10240
ROWS_PER_TILE = NP // 16
CHUNK = 128
GW = 256
DCHUNK = 32
SW = 16
BN = 400
BE = 1000

_PREC = lax.Precision.DEFAULT


def _dot(a, b):
    return jnp.dot(a, b, preferred_element_type=jnp.float32, precision=_PREC)


def _silu(x):
    return x * jax.nn.sigmoid(x)


def _prep_body(h_ref, w1a_ref, w1b_ref, coord_ref, quat_ref,
               ta_ref, tb_ref):
    hb = h_ref[...]
    c = coord_ref[...]
    q = quat_ref[...]
    ss = jnp.sum(q * q, axis=1, keepdims=True)
    qinv = jnp.concatenate([-q[:, :3], q[:, 3:4]], axis=1) / ss
    pad = jnp.zeros((c.shape[0], 5 + GW - 144), jnp.float32)
    rec = jnp.concatenate([c, q, qinv, pad], axis=1)
    ta_ref[...] = jnp.concatenate([_dot(hb, w1a_ref[...]), rec], axis=1)
    tb_ref[...] = jnp.concatenate([_dot(hb, w1b_ref[...]), rec], axis=1)


def _prep(h, w1a, w1b, coord, quat):
    grid = NN // BN
    return pl.pallas_call(
        _prep_body,
        grid=(grid,),
        in_specs=[
            pl.BlockSpec((BN, DD), lambda i: (i, 0)),
            pl.BlockSpec((DD, DD), lambda i: (0, 0)),
            pl.BlockSpec((DD, DD), lambda i: (0, 0)),
            pl.BlockSpec((BN, 3), lambda i: (i, 0)),
            pl.BlockSpec((BN, 4), lambda i: (i, 0)),
        ],
        out_specs=[
            pl.BlockSpec((BN, GW), lambda i: (i, 0)),
            pl.BlockSpec((BN, GW), lambda i: (i, 0)),
        ],
        out_shape=[
            jax.ShapeDtypeStruct((NN, GW), jnp.float32),
            jax.ShapeDtypeStruct((NN, GW), jnp.float32),
        ],
    )(h, w1a, w1b, coord, quat)


def _sc_mesh():
    return plsc.VectorSubcoreMesh(core_axis_name="c", subcore_axis_name="s")


def _gather_one(table, idx2d, es):
    @functools.partial(
        pl.kernel,
        out_type=jax.ShapeDtypeStruct((es, GW), jnp.float32),
        mesh=_sc_mesh(),
    )
    def k(t_hbm, idx_hbm, g_hbm):
        def body(i_v, g_v):
            pltpu.sync_copy(t_hbm.at[i_v.at[0]], g_v)

        pltpu.emit_pipeline(
            body,
            grid=(es // CHUNK,),
            in_specs=[pl.BlockSpec((1, CHUNK), lambda i: (i, 0))],
            out_specs=[pl.BlockSpec((CHUNK, GW), lambda i: (i, 0))],
            core_axis_name=("c", "s"),
            dimension_semantics=(pltpu.PARALLEL,),
        )(idx_hbm, g_hbm)

    return k(table, idx2d)


def _dg0(a, b):
    return lax.dot_general(a, b, (((0,), (0,)), ((), ())),
                           preferred_element_type=jnp.float32,
                           precision=_PREC)


def _dgT(w, x):
    return lax.dot_general(w, x, (((0,), (1,)), ((), ())),
                           preferred_element_type=jnp.float32,
                           precision=_PREC)


def _edge_body(ga_ref, gb_ref, ea_ref, sela_ref, selb_ref, p8_ref,
               w1g_ref, w1e_ref, eb1_ref, ew2_ref, eb2_ref,
               cw1_ref, cb1_ref, cw2_ref, qw1_ref, qb1_ref, qw2_ref, qb2_ref,
               m_ref, s_ref):
    GA = ga_ref[...]
    GB = gb_ref[...]
    At = _dgT(sela_ref[...], GA)
    Bt = _dgT(selb_ref[...], GB)
    cdx = At[0:1] - Bt[0:1]
    cdy = At[1:2] - Bt[1:2]
    cdz = At[2:3] - Bt[2:3]
    radial = cdx * cdx + cdy * cdy + cdz * cdz
    inv = 1.0 / (jnp.sqrt(radial) + EPS)
    px, py, pz, pw = At[3:4], At[4:5], At[5:6], At[6:7]
    qx, qy, qz, qw = Bt[3:4], Bt[4:5], Bt[5:6], Bt[6:7]
    vx = pw * qx + qw * px + (py * qz - pz * qy)
    vy = pw * qy + qw * py + (pz * qx - px * qz)
    vz = pw * qz + qw * pz + (px * qy - py * qx)
    vw = pw * qw - (px * qx + py * qy + pz * qz)
    dx, dy, dz = cdx * inv, cdy * inv, cdz * inv
    tx = 2.0 * (py * dz - pz * dy)
    ty = 2.0 * (pz * dx - px * dz)
    tz = 2.0 * (px * dy - py * dx)
    ux = -(dx + pw * tx + (py * tz - pz * ty))
    uy = -(dy + pw * ty + (pz * tx - px * tz))
    uz = -(dz + pw * tz + (px * ty - py * tx))
    geomT = jnp.concatenate([radial, vx, vy, vz, vw, ux, uy, uz], axis=0)
    m1 = _silu(GA[:, :DD] + GB[:, :DD] + _dg0(geomT, w1g_ref[...])
               + _dot(ea_ref[...], w1e_ref[...]) + eb1_ref[...])
    m = _silu(_dot(m1, ew2_ref[...]) + eb2_ref[...])
    ch = _silu(_dot(m, cw1_ref[...]) + cb1_ref[...])
    cmT = _dgT(cw2_ref[...], ch)
    th = _silu(_dot(m, qw1_ref[...]) + qb1_ref[...])
    tvT = _dgT(qw2_ref[...], th) + qb2_ref[...]
    m_ref[...] = m
    sT = jnp.concatenate(
        [cdx * cmT, cdy * cmT, cdz * cmT, tvT,
         jnp.ones((1, cmT.shape[1]), jnp.float32),
         jnp.zeros((1, cmT.shape[1]), jnp.float32)], axis=0)
    s_ref[...] = _dg0(sT, p8_ref[...])


def _edge(ga, gb, ea, sela, selb, p8, w1g, w1e, eb1, ew2, eb2,
          cw1, cb1, cw2, qw1, qb1, qw2, qb2):
    grid = ga.shape[0] // BE
    full = lambda r, c: pl.BlockSpec((r, c), lambda i: (0, 0))
    blk = lambda r, c: pl.BlockSpec((r, c), lambda i: (i, 0))
    return pl.pallas_call(
        _edge_body,
        grid=(grid,),
        in_specs=[
            blk(BE, GW), blk(BE, GW), blk(BE, 16),
            full(GW, 8), full(GW, 8), full(8, DD),
            full(8, DD), full(16, DD), full(1, DD), full(DD, DD), full(1, DD),
            full(DD, DD), full(1, DD), full(DD, 1),
            full(DD, DD), full(1, DD), full(DD, 3), full(3, 1),
        ],
        out_specs=[blk(BE, DD), blk(BE, DD)],
        out_shape=[
            jax.ShapeDtypeStruct((ga.shape[0], DD), jnp.float32),
            jax.ShapeDtypeStruct((ga.shape[0], DD), jnp.float32),
        ],
    )(ga, gb, ea, sela, selb, p8, w1g, w1e, eb1, ew2, eb2,
      cw1, cb1, cw2, qw1, qb1, qw2, qb2)


def _scatter_one(data, row2d, zeros_init, width):
    es = data.shape[0]
    @functools.partial(
        pl.kernel,
        out_type=jax.ShapeDtypeStruct((2, NP, width), jnp.float32),
        mesh=_sc_mesh(),
        scratch_types=[pltpu.VMEM_SHARED((NP, width), jnp.float32),
                       pltpu.VMEM((CHUNK,), jnp.int32),
                       pltpu.VMEM((CHUNK, width), jnp.float32)],
    )
    def k(d_hbm, row_hbm, z_hbm, p_hbm, acc, idx1d, dbuf):
        cid = lax.axis_index("c")
        sid = lax.axis_index("s")
        wid = sid * 2 + cid
        r0 = sid * ROWS_PER_TILE

        @pl.loop(0, ROWS_PER_TILE, step=DCHUNK)
        def _(j):
            pltpu.sync_copy(z_hbm.at[pl.ds(r0 + j, DCHUNK)],
                            acc.at[pl.ds(r0 + j, DCHUNK)])

        plsc.subcore_barrier()

        n_chunks = es // CHUNK

        @pl.loop(0, (n_chunks + 31) // 32)
        def _(t):
            chunk = wid + t * 32

            @pl.when(chunk < n_chunks)
            def _():
                pltpu.sync_copy(row_hbm.at[chunk], idx1d)
                pltpu.sync_copy(d_hbm.at[pl.ds(chunk * CHUNK, CHUNK)], dbuf)
                pltpu.sync_copy(dbuf, acc.at[idx1d], add=True)

        plsc.subcore_barrier()

        @pl.loop(0, ROWS_PER_TILE, step=DCHUNK)
        def _(j):
            pltpu.sync_copy(acc.at[pl.ds(r0 + j, DCHUNK)],
                            p_hbm.at[cid, pl.ds(r0 + j, DCHUNK)])

    return k(data, row2d, zeros_init)


def _node_body(h_ref, coord_ref, quat_ref, ph_ref, ps_ref,
               nw1a_ref, nw1b_ref, nb1_ref, nw2_ref, nb2_ref,
               h_out_ref, coord_out_ref, quat_out_ref):
    agg_h = jnp.sum(ph_ref[...], axis=0)
    agg_s = jnp.sum(ps_ref[...], axis=0)
    cnt = jnp.clip(agg_s[:, 6:7], 1.0)
    coord_out_ref[...] = coord_ref[...] + agg_s[:, 0:3] / cnt
    mtx = agg_s[:, 3:4] / cnt
    mty = agg_s[:, 4:5] / cnt
    mtz = agg_s[:, 5:6] / cnt
    r = jnp.sqrt(mtx * mtx + mty * mty + mtz * mtz)
    sr = jnp.sin(r) / r
    qx, qy, qz, qw = mtx * sr, mty * sr, mtz * sr, jnp.cos(r)
    q = quat_ref[...]
    px, py, pz, pw = q[:, 0:1], q[:, 1:2], q[:, 2:3], q[:, 3:4]
    ox = pw * qx + qw * px + (py * qz - pz * qy)
    oy = pw * qy + qw * py + (pz * qx - px * qz)
    oz = pw * qz + qw * pz + (px * qy - py * qx)
    ow = pw * qw - (px * qx + py * qy + pz * qz)
    quat_out_ref[...] = jnp.concatenate([ox, oy, oz, ow], axis=1)
    hb = h_ref[...]
    hid = _silu(_dot(hb, nw1a_ref[...]) + _dot(agg_h, nw1b_ref[...])
                + nb1_ref[...])
    h_out_ref[...] = hb + _dot(hid, nw2_ref[...]) + nb2_ref[...]


def _node(h, coord, quat, ph, ps, nw1a, nw1b, nb1, nw2, nb2):
    grid = NN // BN
    kk = ph.shape[0]
    full = lambda r, c: pl.BlockSpec((r, c), lambda i: (0, 0))
    return pl.pallas_call(
        _node_body,
        grid=(grid,),
        in_specs=[
            pl.BlockSpec((BN, DD), lambda i: (i, 0)),
            pl.BlockSpec((BN, 3), lambda i: (i, 0)),
            pl.BlockSpec((BN, 4), lambda i: (i, 0)),
            pl.BlockSpec((kk, BN, DD), lambda i: (0, i, 0)),
            pl.BlockSpec((kk, BN, DD), lambda i: (0, i, 0)),
            full(DD, DD), full(DD, DD), full(1, DD), full(DD, DD), full(1, DD),
        ],
        out_specs=[
            pl.BlockSpec((BN, DD), lambda i: (i, 0)),
            pl.BlockSpec((BN, 3), lambda i: (i, 0)),
            pl.BlockSpec((BN, 4), lambda i: (i, 0)),
        ],
        out_shape=[
            jax.ShapeDtypeStruct((NN, DD), jnp.float32),
            jax.ShapeDtypeStruct((NN, 3), jnp.float32),
            jax.ShapeDtypeStruct((NN, 4), jnp.float32),
        ],
    )(h, coord, quat, ph, ps, nw1a, nw1b, nb1, nw2, nb2)


def kernel(h, edge_index, coord, quat, edge_attr,
           ew1, eb1, ew2, eb2, nw1, nb1, nw2, nb2,
           cw1, cb1, cw2, qw1, qb1, qw2, qb2):
    row2d = edge_index[0].reshape(EE // CHUNK, CHUNK)
    col2d = edge_index[1].reshape(EE // CHUNK, CHUNK)

    ta, tb = _prep(h, ew1[0:128], ew1[128:256], coord, quat)
    import numpy as _np
    _sela = _np.zeros((GW, 8), _np.float32)
    _selb = _np.zeros((GW, 8), _np.float32)
    for _i, _c in enumerate([128, 129, 130, 135, 136, 137, 138]):
        _sela[_c, _i] = 1.0
    for _i, _c in enumerate([128, 129, 130, 131, 132, 133, 134]):
        _selb[_c, _i] = 1.0
    _p8 = _np.zeros((8, DD), _np.float32)
    for _i in range(7):
        _p8[_i, _i] = 1.0
    zh = jnp.zeros((NP, DD), jnp.float32)
    nch = EE // CHUNK
    phs, pss = [], []
    nsl = 4
    for sl in range(nsl):
        r2 = lax.slice_in_dim(row2d, sl * (nch // nsl), (sl + 1) * (nch // nsl), axis=0)
        c2 = lax.slice_in_dim(col2d, sl * (nch // nsl), (sl + 1) * (nch // nsl), axis=0)
        ea = lax.slice_in_dim(edge_attr, sl * (EE // nsl), (sl + 1) * (EE // nsl), axis=0)
        ga = _gather_one(ta, r2, EE // nsl)
        gb = _gather_one(tb, c2, EE // nsl)
        m, s = _edge(
            ga, gb, ea,
            jnp.asarray(_sela), jnp.asarray(_selb), jnp.asarray(_p8),
            ew1[256:264], ew1[264:280], eb1.reshape(1, DD),
            ew2, eb2.reshape(1, DD),
            cw1, cb1.reshape(1, DD), cw2,
            qw1, qb1.reshape(1, DD), qw2, qb2.reshape(3, 1),
        )
        phs.append(_scatter_one(m, r2, zh, DD))
        pss.append(_scatter_one(s, r2, zh, DD))
    h_out, coord_out, quat_out = _node(
        h, coord, quat, jnp.concatenate(phs, axis=0), jnp.concatenate(pss, axis=0),
        nw1[0:128], nw1[128:256], nb1.reshape(1, DD), nw2,
        nb2.reshape(1, DD))
    return (h_out, coord_out, quat_out, edge_attr)


# speedup vs baseline: 1.0128x; 1.0128x over previous
import functools

import jax
import jax.numpy as jnp
from jax import lax
from jax.experimental import pallas as pl
from jax.experimental.pallas import tpu as pltpu
from jax.experimental.pallas import tpu_sc as plsc

NN = 10000
EE = 320000
DD = 128
EPS = 1e-8
NP = 10240
ROWS_PER_TILE = NP // 16
CHUNK = 128
GW = 256
DCHUNK = 32
SW = 16
BN = 400
BE = 1000

_PREC = lax.Precision.DEFAULT


def _dot(a, b):
    return jnp.dot(a, b, preferred_element_type=jnp.float32, precision=_PREC)


def _silu(x):
    return x * jax.nn.sigmoid(x)


def _prep_body(h_ref, w1a_ref, w1b_ref, coord_ref, quat_ref,
               ta_ref, tb_ref):
    hb = h_ref[...]
    c = coord_ref[...]
    q = quat_ref[...]
    ss = jnp.sum(q * q, axis=1, keepdims=True)
    qinv = jnp.concatenate([-q[:, :3], q[:, 3:4]], axis=1) / ss
    pad = jnp.zeros((c.shape[0], 5 + GW - 144), jnp.float32)
    rec = jnp.concatenate([c, q, qinv, pad], axis=1)
    ta_ref[...] = jnp.concatenate([_dot(hb, w1a_ref[...]), rec], axis=1)
    tb_ref[...] = jnp.concatenate([_dot(hb, w1b_ref[...]), rec], axis=1)


def _prep(h, w1a, w1b, coord, quat):
    grid = NN // BN
    return pl.pallas_call(
        _prep_body,
        grid=(grid,),
        in_specs=[
            pl.BlockSpec((BN, DD), lambda i: (i, 0)),
            pl.BlockSpec((DD, DD), lambda i: (0, 0)),
            pl.BlockSpec((DD, DD), lambda i: (0, 0)),
            pl.BlockSpec((BN, 3), lambda i: (i, 0)),
            pl.BlockSpec((BN, 4), lambda i: (i, 0)),
        ],
        out_specs=[
            pl.BlockSpec((BN, GW), lambda i: (i, 0)),
            pl.BlockSpec((BN, GW), lambda i: (i, 0)),
        ],
        out_shape=[
            jax.ShapeDtypeStruct((NN, GW), jnp.float32),
            jax.ShapeDtypeStruct((NN, GW), jnp.float32),
        ],
    )(h, w1a, w1b, coord, quat)


def _sc_mesh():
    return plsc.VectorSubcoreMesh(core_axis_name="c", subcore_axis_name="s")


def _gather_one(table, idx2d, es):
    @functools.partial(
        pl.kernel,
        out_type=jax.ShapeDtypeStruct((es, GW), jnp.float32),
        mesh=_sc_mesh(),
    )
    def k(t_hbm, idx_hbm, g_hbm):
        def body(i_v, g_v):
            pltpu.sync_copy(t_hbm.at[i_v.at[0]], g_v)

        pltpu.emit_pipeline(
            body,
            grid=(es // CHUNK,),
            in_specs=[pl.BlockSpec((1, CHUNK), lambda i: (i, 0))],
            out_specs=[pl.BlockSpec((CHUNK, GW), lambda i: (i, 0))],
            core_axis_name=("c", "s"),
            dimension_semantics=(pltpu.PARALLEL,),
        )(idx_hbm, g_hbm)

    return k(table, idx2d)


def _dg0(a, b):
    return lax.dot_general(a, b, (((0,), (0,)), ((), ())),
                           preferred_element_type=jnp.float32,
                           precision=_PREC)


def _dgT(w, x):
    return lax.dot_general(w, x, (((0,), (1,)), ((), ())),
                           preferred_element_type=jnp.float32,
                           precision=_PREC)


def _edge_body(ga_ref, gb_ref, ea_ref, sela_ref, selb_ref, p8_ref,
               w1g_ref, w1e_ref, eb1_ref, ew2_ref, eb2_ref,
               cw1_ref, cb1_ref, cw2_ref, qw1_ref, qb1_ref, qw2_ref, qb2_ref,
               m_ref, s_ref):
    GA = ga_ref[...]
    GB = gb_ref[...]
    At = _dgT(sela_ref[...], GA)
    Bt = _dgT(selb_ref[...], GB)
    cdx = At[0:1] - Bt[0:1]
    cdy = At[1:2] - Bt[1:2]
    cdz = At[2:3] - Bt[2:3]
    radial = cdx * cdx + cdy * cdy + cdz * cdz
    inv = 1.0 / (jnp.sqrt(radial) + EPS)
    px, py, pz, pw = At[3:4], At[4:5], At[5:6], At[6:7]
    qx, qy, qz, qw = Bt[3:4], Bt[4:5], Bt[5:6], Bt[6:7]
    vx = pw * qx + qw * px + (py * qz - pz * qy)
    vy = pw * qy + qw * py + (pz * qx - px * qz)
    vz = pw * qz + qw * pz + (px * qy - py * qx)
    vw = pw * qw - (px * qx + py * qy + pz * qz)
    dx, dy, dz = cdx * inv, cdy * inv, cdz * inv
    tx = 2.0 * (py * dz - pz * dy)
    ty = 2.0 * (pz * dx - px * dz)
    tz = 2.0 * (px * dy - py * dx)
    ux = -(dx + pw * tx + (py * tz - pz * ty))
    uy = -(dy + pw * ty + (pz * tx - px * tz))
    uz = -(dz + pw * tz + (px * ty - py * tx))
    geomT = jnp.concatenate([radial, vx, vy, vz, vw, ux, uy, uz], axis=0)
    m1 = _silu(GA[:, :DD] + GB[:, :DD] + _dg0(geomT, w1g_ref[...])
               + _dot(ea_ref[...], w1e_ref[...]) + eb1_ref[...])
    m = _silu(_dot(m1, ew2_ref[...]) + eb2_ref[...])
    ch = _silu(_dot(m, cw1_ref[...]) + cb1_ref[...])
    cmT = _dgT(cw2_ref[...], ch)
    th = _silu(_dot(m, qw1_ref[...]) + qb1_ref[...])
    tvT = _dgT(qw2_ref[...], th) + qb2_ref[...]
    m_ref[...] = m
    sT = jnp.concatenate(
        [cdx * cmT, cdy * cmT, cdz * cmT, tvT,
         jnp.ones((1, cmT.shape[1]), jnp.float32),
         jnp.zeros((1, cmT.shape[1]), jnp.float32)], axis=0)
    s_ref[...] = _dg0(sT, p8_ref[...])


def _edge(ga, gb, ea, sela, selb, p8, w1g, w1e, eb1, ew2, eb2,
          cw1, cb1, cw2, qw1, qb1, qw2, qb2):
    grid = ga.shape[0] // BE
    full = lambda r, c: pl.BlockSpec((r, c), lambda i: (0, 0))
    blk = lambda r, c: pl.BlockSpec((r, c), lambda i: (i, 0))
    return pl.pallas_call(
        _edge_body,
        grid=(grid,),
        in_specs=[
            blk(BE, GW), blk(BE, GW), blk(BE, 16),
            full(GW, 8), full(GW, 8), full(8, DD),
            full(8, DD), full(16, DD), full(1, DD), full(DD, DD), full(1, DD),
            full(DD, DD), full(1, DD), full(DD, 1),
            full(DD, DD), full(1, DD), full(DD, 3), full(3, 1),
        ],
        out_specs=[blk(BE, DD), blk(BE, DD)],
        out_shape=[
            jax.ShapeDtypeStruct((ga.shape[0], DD), jnp.float32),
            jax.ShapeDtypeStruct((ga.shape[0], DD), jnp.float32),
        ],
    )(ga, gb, ea, sela, selb, p8, w1g, w1e, eb1, ew2, eb2,
      cw1, cb1, cw2, qw1, qb1, qw2, qb2)


def _scatter_one(data, row2d, zeros_init, width):
    es = data.shape[0]
    @functools.partial(
        pl.kernel,
        out_type=jax.ShapeDtypeStruct((2, NP, width), jnp.float32),
        mesh=_sc_mesh(),
        scratch_types=[pltpu.VMEM_SHARED((NP, width), jnp.float32),
                       pltpu.VMEM((CHUNK,), jnp.int32),
                       pltpu.VMEM((CHUNK, width), jnp.float32)],
    )
    def k(d_hbm, row_hbm, z_hbm, p_hbm, acc, idx1d, dbuf):
        cid = lax.axis_index("c")
        sid = lax.axis_index("s")
        wid = sid * 2 + cid
        r0 = sid * ROWS_PER_TILE

        @pl.loop(0, ROWS_PER_TILE, step=DCHUNK)
        def _(j):
            pltpu.sync_copy(z_hbm.at[pl.ds(r0 + j, DCHUNK)],
                            acc.at[pl.ds(r0 + j, DCHUNK)])

        plsc.subcore_barrier()

        n_chunks = es // CHUNK

        @pl.loop(0, (n_chunks + 31) // 32)
        def _(t):
            chunk = wid + t * 32

            @pl.when(chunk < n_chunks)
            def _():
                pltpu.sync_copy(row_hbm.at[chunk], idx1d)
                pltpu.sync_copy(d_hbm.at[pl.ds(chunk * CHUNK, CHUNK)], dbuf)
                pltpu.sync_copy(dbuf, acc.at[idx1d], add=True)

        plsc.subcore_barrier()

        @pl.loop(0, ROWS_PER_TILE, step=DCHUNK)
        def _(j):
            pltpu.sync_copy(acc.at[pl.ds(r0 + j, DCHUNK)],
                            p_hbm.at[cid, pl.ds(r0 + j, DCHUNK)])

    return k(data, row2d, zeros_init)


def _node_body(h_ref, coord_ref, quat_ref, ph_ref, ps_ref,
               nw1a_ref, nw1b_ref, nb1_ref, nw2_ref, nb2_ref,
               h_out_ref, coord_out_ref, quat_out_ref):
    agg_h = jnp.sum(ph_ref[...], axis=0)
    agg_s = jnp.sum(ps_ref[...], axis=0)
    cnt = jnp.clip(agg_s[:, 6:7], 1.0)
    coord_out_ref[...] = coord_ref[...] + agg_s[:, 0:3] / cnt
    mtx = agg_s[:, 3:4] / cnt
    mty = agg_s[:, 4:5] / cnt
    mtz = agg_s[:, 5:6] / cnt
    r = jnp.sqrt(mtx * mtx + mty * mty + mtz * mtz)
    sr = jnp.sin(r) / r
    qx, qy, qz, qw = mtx * sr, mty * sr, mtz * sr, jnp.cos(r)
    q = quat_ref[...]
    px, py, pz, pw = q[:, 0:1], q[:, 1:2], q[:, 2:3], q[:, 3:4]
    ox = pw * qx + qw * px + (py * qz - pz * qy)
    oy = pw * qy + qw * py + (pz * qx - px * qz)
    oz = pw * qz + qw * pz + (px * qy - py * qx)
    ow = pw * qw - (px * qx + py * qy + pz * qz)
    quat_out_ref[...] = jnp.concatenate([ox, oy, oz, ow], axis=1)
    hb = h_ref[...]
    hid = _silu(_dot(hb, nw1a_ref[...]) + _dot(agg_h, nw1b_ref[...])
                + nb1_ref[...])
    h_out_ref[...] = hb + _dot(hid, nw2_ref[...]) + nb2_ref[...]


def _node(h, coord, quat, ph, ps, nw1a, nw1b, nb1, nw2, nb2):
    grid = NN // BN
    kk = ph.shape[0]
    full = lambda r, c: pl.BlockSpec((r, c), lambda i: (0, 0))
    return pl.pallas_call(
        _node_body,
        grid=(grid,),
        in_specs=[
            pl.BlockSpec((BN, DD), lambda i: (i, 0)),
            pl.BlockSpec((BN, 3), lambda i: (i, 0)),
            pl.BlockSpec((BN, 4), lambda i: (i, 0)),
            pl.BlockSpec((kk, BN, DD), lambda i: (0, i, 0)),
            pl.BlockSpec((kk, BN, DD), lambda i: (0, i, 0)),
            full(DD, DD), full(DD, DD), full(1, DD), full(DD, DD), full(1, DD),
        ],
        out_specs=[
            pl.BlockSpec((BN, DD), lambda i: (i, 0)),
            pl.BlockSpec((BN, 3), lambda i: (i, 0)),
            pl.BlockSpec((BN, 4), lambda i: (i, 0)),
        ],
        out_shape=[
            jax.ShapeDtypeStruct((NN, DD), jnp.float32),
            jax.ShapeDtypeStruct((NN, 3), jnp.float32),
            jax.ShapeDtypeStruct((NN, 4), jnp.float32),
        ],
    )(h, coord, quat, ph, ps, nw1a, nw1b, nb1, nw2, nb2)


def kernel(h, edge_index, coord, quat, edge_attr,
           ew1, eb1, ew2, eb2, nw1, nb1, nw2, nb2,
           cw1, cb1, cw2, qw1, qb1, qw2, qb2):
    row2d = edge_index[0].reshape(EE // CHUNK, CHUNK)
    col2d = edge_index[1].reshape(EE // CHUNK, CHUNK)

    ta, tb = _prep(h, ew1[0:128], ew1[128:256], coord, quat)
    import numpy as _np
    _sela = _np.zeros((GW, 8), _np.float32)
    _selb = _np.zeros((GW, 8), _np.float32)
    for _i, _c in enumerate([128, 129, 130, 135, 136, 137, 138]):
        _sela[_c, _i] = 1.0
    for _i, _c in enumerate([128, 129, 130, 131, 132, 133, 134]):
        _selb[_c, _i] = 1.0
    _p8 = _np.zeros((8, DD), _np.float32)
    for _i in range(7):
        _p8[_i, _i] = 1.0
    zh = jnp.zeros((NP, DD), jnp.float32)
    nch = EE // CHUNK
    phs, pss = [], []
    nsl = 2
    for sl in range(nsl):
        r2 = lax.slice_in_dim(row2d, sl * (nch // nsl), (sl + 1) * (nch // nsl), axis=0)
        c2 = lax.slice_in_dim(col2d, sl * (nch // nsl), (sl + 1) * (nch // nsl), axis=0)
        ea = lax.slice_in_dim(edge_attr, sl * (EE // nsl), (sl + 1) * (EE // nsl), axis=0)
        ga = _gather_one(ta, r2, EE // nsl)
        gb = _gather_one(tb, c2, EE // nsl)
        m, s = _edge(
            ga, gb, ea,
            jnp.asarray(_sela), jnp.asarray(_selb), jnp.asarray(_p8),
            ew1[256:264], ew1[264:280], eb1.reshape(1, DD),
            ew2, eb2.reshape(1, DD),
            cw1, cb1.reshape(1, DD), cw2,
            qw1, qb1.reshape(1, DD), qw2, qb2.reshape(3, 1),
        )
        phs.append(_scatter_one(m, r2, zh, DD))
        pss.append(_scatter_one(s, r2, zh, DD))
    h_out, coord_out, quat_out = _node(
        h, coord, quat, jnp.concatenate(phs, axis=0), jnp.concatenate(pss, axis=0),
        nw1[0:128], nw1[128:256], nb1.reshape(1, DD), nw2,
        nb2.reshape(1, DD))
    return (h_out, coord_out, quat_out, edge_attr)


# speedup vs baseline: 1.0273x; 1.0143x over previous
import functools

import jax
import jax.numpy as jnp
from jax import lax
from jax.experimental import pallas as pl
from jax.experimental.pallas import tpu as pltpu
from jax.experimental.pallas import tpu_sc as plsc

NN = 10000
EE = 320000
DD = 128
EPS = 1e-8
NP = 10240
ROWS_PER_TILE = NP // 16
CHUNK = 128
GW = 256
DCHUNK = 32
SW = 16
BN = 400
BE = 1000

_PREC = lax.Precision.DEFAULT


def _dot(a, b):
    return jnp.dot(a, b, preferred_element_type=jnp.float32, precision=_PREC)


def _silu(x):
    return x * jax.nn.sigmoid(x)


def _prep_body(h_ref, w1a_ref, w1b_ref, coord_ref, quat_ref,
               ta_ref, tb_ref):
    hb = h_ref[...]
    c = coord_ref[...]
    q = quat_ref[...]
    ss = jnp.sum(q * q, axis=1, keepdims=True)
    qinv = jnp.concatenate([-q[:, :3], q[:, 3:4]], axis=1) / ss
    pad = jnp.zeros((c.shape[0], 5 + GW - 144), jnp.float32)
    rec = jnp.concatenate([c, q, qinv, pad], axis=1)
    ta_ref[...] = jnp.concatenate([_dot(hb, w1a_ref[...]), rec], axis=1)
    tb_ref[...] = jnp.concatenate([_dot(hb, w1b_ref[...]), rec], axis=1)


def _prep(h, w1a, w1b, coord, quat):
    grid = NN // BN
    return pl.pallas_call(
        _prep_body,
        grid=(grid,),
        in_specs=[
            pl.BlockSpec((BN, DD), lambda i: (i, 0)),
            pl.BlockSpec((DD, DD), lambda i: (0, 0)),
            pl.BlockSpec((DD, DD), lambda i: (0, 0)),
            pl.BlockSpec((BN, 3), lambda i: (i, 0)),
            pl.BlockSpec((BN, 4), lambda i: (i, 0)),
        ],
        out_specs=[
            pl.BlockSpec((BN, GW), lambda i: (i, 0)),
            pl.BlockSpec((BN, GW), lambda i: (i, 0)),
        ],
        out_shape=[
            jax.ShapeDtypeStruct((NN, GW), jnp.float32),
            jax.ShapeDtypeStruct((NN, GW), jnp.float32),
        ],
    )(h, w1a, w1b, coord, quat)


def _sc_mesh():
    return plsc.VectorSubcoreMesh(core_axis_name="c", subcore_axis_name="s")


def _gather_one(table, idx2d, es):
    @functools.partial(
        pl.kernel,
        out_type=jax.ShapeDtypeStruct((es, GW), jnp.float32),
        mesh=_sc_mesh(),
    )
    def k(t_hbm, idx_hbm, g_hbm):
        def body(i_v, g_v):
            pltpu.sync_copy(t_hbm.at[i_v.at[0]], g_v)

        pltpu.emit_pipeline(
            body,
            grid=(es // CHUNK,),
            in_specs=[pl.BlockSpec((1, CHUNK), lambda i: (i, 0))],
            out_specs=[pl.BlockSpec((CHUNK, GW), lambda i: (i, 0))],
            core_axis_name=("c", "s"),
            dimension_semantics=(pltpu.PARALLEL,),
        )(idx_hbm, g_hbm)

    return k(table, idx2d)


def _dg0(a, b):
    return lax.dot_general(a, b, (((0,), (0,)), ((), ())),
                           preferred_element_type=jnp.float32,
                           precision=_PREC)


def _dgT(w, x):
    return lax.dot_general(w, x, (((0,), (1,)), ((), ())),
                           preferred_element_type=jnp.float32,
                           precision=_PREC)


def _edge_body(ga_ref, gb_ref, ea_ref, sela_ref, selb_ref, p8_ref,
               w1g_ref, w1e_ref, eb1_ref, ew2_ref, eb2_ref,
               cw1_ref, cb1_ref, cw2_ref, qw1_ref, qb1_ref, qw2_ref, qb2_ref,
               m_ref, s_ref):
    GA = ga_ref[...]
    GB = gb_ref[...]
    At = _dgT(sela_ref[...], GA)
    Bt = _dgT(selb_ref[...], GB)
    cdx = At[0:1] - Bt[0:1]
    cdy = At[1:2] - Bt[1:2]
    cdz = At[2:3] - Bt[2:3]
    radial = cdx * cdx + cdy * cdy + cdz * cdz
    inv = 1.0 / (jnp.sqrt(radial) + EPS)
    px, py, pz, pw = At[3:4], At[4:5], At[5:6], At[6:7]
    qx, qy, qz, qw = Bt[3:4], Bt[4:5], Bt[5:6], Bt[6:7]
    vx = pw * qx + qw * px + (py * qz - pz * qy)
    vy = pw * qy + qw * py + (pz * qx - px * qz)
    vz = pw * qz + qw * pz + (px * qy - py * qx)
    vw = pw * qw - (px * qx + py * qy + pz * qz)
    dx, dy, dz = cdx * inv, cdy * inv, cdz * inv
    tx = 2.0 * (py * dz - pz * dy)
    ty = 2.0 * (pz * dx - px * dz)
    tz = 2.0 * (px * dy - py * dx)
    ux = -(dx + pw * tx + (py * tz - pz * ty))
    uy = -(dy + pw * ty + (pz * tx - px * tz))
    uz = -(dz + pw * tz + (px * ty - py * tx))
    geomT = jnp.concatenate([radial, vx, vy, vz, vw, ux, uy, uz], axis=0)
    m1 = _silu(GA[:, :DD] + GB[:, :DD] + _dg0(geomT, w1g_ref[...])
               + _dot(ea_ref[...], w1e_ref[...]) + eb1_ref[...])
    m = _silu(_dot(m1, ew2_ref[...]) + eb2_ref[...])
    ch = _silu(_dot(m, cw1_ref[...]) + cb1_ref[...])
    cmT = _dgT(cw2_ref[...], ch)
    th = _silu(_dot(m, qw1_ref[...]) + qb1_ref[...])
    tvT = _dgT(qw2_ref[...], th) + qb2_ref[...]
    m_ref[...] = m
    sT = jnp.concatenate(
        [cdx * cmT, cdy * cmT, cdz * cmT, tvT,
         jnp.ones((1, cmT.shape[1]), jnp.float32),
         jnp.zeros((1, cmT.shape[1]), jnp.float32)], axis=0)
    s_ref[...] = _dg0(sT, p8_ref[...])


def _edge(ga, gb, ea, sela, selb, p8, w1g, w1e, eb1, ew2, eb2,
          cw1, cb1, cw2, qw1, qb1, qw2, qb2):
    grid = ga.shape[0] // BE
    full = lambda r, c: pl.BlockSpec((r, c), lambda i: (0, 0))
    blk = lambda r, c: pl.BlockSpec((r, c), lambda i: (i, 0))
    return pl.pallas_call(
        _edge_body,
        grid=(grid,),
        in_specs=[
            blk(BE, GW), blk(BE, GW), blk(BE, 16),
            full(GW, 8), full(GW, 8), full(8, DD),
            full(8, DD), full(16, DD), full(1, DD), full(DD, DD), full(1, DD),
            full(DD, DD), full(1, DD), full(DD, 1),
            full(DD, DD), full(1, DD), full(DD, 3), full(3, 1),
        ],
        out_specs=[blk(BE, DD), blk(BE, DD)],
        out_shape=[
            jax.ShapeDtypeStruct((ga.shape[0], DD), jnp.float32),
            jax.ShapeDtypeStruct((ga.shape[0], DD), jnp.float32),
        ],
    )(ga, gb, ea, sela, selb, p8, w1g, w1e, eb1, ew2, eb2,
      cw1, cb1, cw2, qw1, qb1, qw2, qb2)


def _scatter_one(data, row2d, zeros_init, width):
    es = data.shape[0]
    @functools.partial(
        pl.kernel,
        out_type=jax.ShapeDtypeStruct((2, NP, width), jnp.float32),
        mesh=_sc_mesh(),
        scratch_types=[pltpu.VMEM_SHARED((NP, width), jnp.float32),
                       pltpu.VMEM((CHUNK,), jnp.int32),
                       pltpu.VMEM((CHUNK, width), jnp.float32)],
    )
    def k(d_hbm, row_hbm, z_hbm, p_hbm, acc, idx1d, dbuf):
        cid = lax.axis_index("c")
        sid = lax.axis_index("s")
        wid = sid * 2 + cid
        r0 = sid * ROWS_PER_TILE

        @pl.loop(0, ROWS_PER_TILE, step=DCHUNK)
        def _(j):
            pltpu.sync_copy(z_hbm.at[pl.ds(r0 + j, DCHUNK)],
                            acc.at[pl.ds(r0 + j, DCHUNK)])

        plsc.subcore_barrier()

        n_chunks = es // CHUNK

        @pl.loop(0, (n_chunks + 31) // 32)
        def _(t):
            chunk = wid + t * 32

            @pl.when(chunk < n_chunks)
            def _():
                pltpu.sync_copy(row_hbm.at[chunk], idx1d)
                pltpu.sync_copy(d_hbm.at[pl.ds(chunk * CHUNK, CHUNK)], dbuf)
                pltpu.sync_copy(dbuf, acc.at[idx1d], add=True)

        plsc.subcore_barrier()

        @pl.loop(0, ROWS_PER_TILE, step=DCHUNK)
        def _(j):
            pltpu.sync_copy(acc.at[pl.ds(r0 + j, DCHUNK)],
                            p_hbm.at[cid, pl.ds(r0 + j, DCHUNK)])

    return k(data, row2d, zeros_init)


def _node_body(h_ref, coord_ref, quat_ref, ph0_ref, ph1_ref,
               ps0_ref, ps1_ref,
               nw1a_ref, nw1b_ref, nb1_ref, nw2_ref, nb2_ref,
               h_out_ref, coord_out_ref, quat_out_ref):
    agg_h = ph0_ref[0] + ph0_ref[1] + ph1_ref[0] + ph1_ref[1]
    agg_s = ps0_ref[0] + ps0_ref[1] + ps1_ref[0] + ps1_ref[1]
    cnt = jnp.clip(agg_s[:, 6:7], 1.0)
    coord_out_ref[...] = coord_ref[...] + agg_s[:, 0:3] / cnt
    mtx = agg_s[:, 3:4] / cnt
    mty = agg_s[:, 4:5] / cnt
    mtz = agg_s[:, 5:6] / cnt
    r = jnp.sqrt(mtx * mtx + mty * mty + mtz * mtz)
    sr = jnp.sin(r) / r
    qx, qy, qz, qw = mtx * sr, mty * sr, mtz * sr, jnp.cos(r)
    q = quat_ref[...]
    px, py, pz, pw = q[:, 0:1], q[:, 1:2], q[:, 2:3], q[:, 3:4]
    ox = pw * qx + qw * px + (py * qz - pz * qy)
    oy = pw * qy + qw * py + (pz * qx - px * qz)
    oz = pw * qz + qw * pz + (px * qy - py * qx)
    ow = pw * qw - (px * qx + py * qy + pz * qz)
    quat_out_ref[...] = jnp.concatenate([ox, oy, oz, ow], axis=1)
    hb = h_ref[...]
    hid = _silu(_dot(hb, nw1a_ref[...]) + _dot(agg_h, nw1b_ref[...])
                + nb1_ref[...])
    h_out_ref[...] = hb + _dot(hid, nw2_ref[...]) + nb2_ref[...]


def _node(h, coord, quat, ph0, ph1, ps0, ps1, nw1a, nw1b, nb1, nw2, nb2):
    grid = NN // BN
    full = lambda r, c: pl.BlockSpec((r, c), lambda i: (0, 0))
    return pl.pallas_call(
        _node_body,
        grid=(grid,),
        in_specs=[
            pl.BlockSpec((BN, DD), lambda i: (i, 0)),
            pl.BlockSpec((BN, 3), lambda i: (i, 0)),
            pl.BlockSpec((BN, 4), lambda i: (i, 0)),
            pl.BlockSpec((2, BN, DD), lambda i: (0, i, 0)),
            pl.BlockSpec((2, BN, DD), lambda i: (0, i, 0)),
            pl.BlockSpec((2, BN, DD), lambda i: (0, i, 0)),
            pl.BlockSpec((2, BN, DD), lambda i: (0, i, 0)),
            full(DD, DD), full(DD, DD), full(1, DD), full(DD, DD), full(1, DD),
        ],
        out_specs=[
            pl.BlockSpec((BN, DD), lambda i: (i, 0)),
            pl.BlockSpec((BN, 3), lambda i: (i, 0)),
            pl.BlockSpec((BN, 4), lambda i: (i, 0)),
        ],
        out_shape=[
            jax.ShapeDtypeStruct((NN, DD), jnp.float32),
            jax.ShapeDtypeStruct((NN, 3), jnp.float32),
            jax.ShapeDtypeStruct((NN, 4), jnp.float32),
        ],
    )(h, coord, quat, ph0, ph1, ps0, ps1, nw1a, nw1b, nb1, nw2, nb2)


def kernel(h, edge_index, coord, quat, edge_attr,
           ew1, eb1, ew2, eb2, nw1, nb1, nw2, nb2,
           cw1, cb1, cw2, qw1, qb1, qw2, qb2):
    row2d = edge_index[0].reshape(EE // CHUNK, CHUNK)
    col2d = edge_index[1].reshape(EE // CHUNK, CHUNK)

    ta, tb = _prep(h, ew1[0:128], ew1[128:256], coord, quat)
    import numpy as _np
    _sela = _np.zeros((GW, 8), _np.float32)
    _selb = _np.zeros((GW, 8), _np.float32)
    for _i, _c in enumerate([128, 129, 130, 135, 136, 137, 138]):
        _sela[_c, _i] = 1.0
    for _i, _c in enumerate([128, 129, 130, 131, 132, 133, 134]):
        _selb[_c, _i] = 1.0
    _p8 = _np.zeros((8, DD), _np.float32)
    for _i in range(7):
        _p8[_i, _i] = 1.0
    zh = jnp.zeros((NP, DD), jnp.float32)
    nch = EE // CHUNK
    phs, pss = [], []
    nsl = 2
    for sl in range(nsl):
        r2 = lax.slice_in_dim(row2d, sl * (nch // nsl), (sl + 1) * (nch // nsl), axis=0)
        c2 = lax.slice_in_dim(col2d, sl * (nch // nsl), (sl + 1) * (nch // nsl), axis=0)
        ea = lax.slice_in_dim(edge_attr, sl * (EE // nsl), (sl + 1) * (EE // nsl), axis=0)
        ga = _gather_one(ta, r2, EE // nsl)
        gb = _gather_one(tb, c2, EE // nsl)
        m, s = _edge(
            ga, gb, ea,
            jnp.asarray(_sela), jnp.asarray(_selb), jnp.asarray(_p8),
            ew1[256:264], ew1[264:280], eb1.reshape(1, DD),
            ew2, eb2.reshape(1, DD),
            cw1, cb1.reshape(1, DD), cw2,
            qw1, qb1.reshape(1, DD), qw2, qb2.reshape(3, 1),
        )
        phs.append(_scatter_one(m, r2, zh, DD))
        pss.append(_scatter_one(s, r2, zh, DD))
    h_out, coord_out, quat_out = _node(
        h, coord, quat, phs[0], phs[1], pss[0], pss[1],
        nw1[0:128], nw1[128:256], nb1.reshape(1, DD), nw2,
        nb2.reshape(1, DD))
    return (h_out, coord_out, quat_out, edge_attr)
